# baseline (speedup 1.0000x reference)
import functools

import jax
import jax.numpy as jnp
from jax import lax
from jax.experimental import pallas as pl
from jax.experimental.pallas import tpu as pltpu
from jax.experimental.pallas import tpu_sc as plsc

N = 10000
E = 320000
D = 128
HD = 64
R = 3
NGRAPH = 16
NSUB = 16
SACC = 30208
HACC = 30720
DUMP = 30000
TPT = SACC // NSUB
EPAD = 327680
ECH = 64
CHUNKS = EPAD // NSUB // ECH
NBUF = 2
ECNT = E // 2 // NSUB
CCH = 80
CNCH = ECNT // CCH
NPAD = 10240
BN = 512
PBN = 400


def _prep_body(dst_hbm, et_hbm, cnt0_hbm, cnt1_hbm, dst3_hbm, dbuf, ebuf,
               obuf, onesb, sumbuf, spm):
    c = lax.axis_index("c")
    s = lax.axis_index("s")
    zz = jnp.zeros((16,), jnp.float32)
    def zero(i, carry):
        sumbuf[pl.ds(i * 16, 16)] = zz
        return carry
    lax.fori_loop(0, TPT // 16, zero, 0)
    pltpu.sync_copy(sumbuf, spm.at[pl.ds(s * TPT, TPT)])
    ones = jnp.ones((16,), jnp.float32)
    for j in range(CCH // 16):
        onesb[pl.ds(j * 16, 16)] = ones
    plsc.subcore_barrier()

    def chunk(k, carry):
        base = c * (E // 2) + s * ECNT + k * CCH
        pltpu.sync_copy(dst_hbm.at[pl.ds(base, CCH)], dbuf)
        pltpu.sync_copy(et_hbm.at[pl.ds(base, CCH)], ebuf)
        for j in range(CCH // 16):
            sl = pl.ds(j * 16, 16)
            obuf[sl] = dbuf[sl] * 3 + ebuf[sl]
        pltpu.sync_copy(onesb, spm.at[obuf], add=True)
        pltpu.sync_copy(obuf, dst3_hbm.at[pl.ds(base, CCH)])
        return carry
    lax.fori_loop(0, CNCH, chunk, 0)

    @pl.when(c == 1)
    def _tail():
        for j in range(CCH // 16):
            obuf[pl.ds(j * 16, 16)] = jnp.full((16,), DUMP, jnp.int32)
        def tail(k, carry):
            pltpu.sync_copy(obuf,
                            dst3_hbm.at[pl.ds(E + (s * 6 + k) * CCH, CCH)])
            return carry
        lax.fori_loop(0, (EPAD - E) // CCH // NSUB, tail, 0)

    plsc.subcore_barrier()
    pltpu.sync_copy(spm.at[pl.ds(s * TPT, TPT)], sumbuf)

    @pl.when(c == 0)
    def _c0():
        pltpu.sync_copy(sumbuf, cnt0_hbm.at[pl.ds(s * TPT, TPT)])

    @pl.when(c == 1)
    def _c1():
        pltpu.sync_copy(sumbuf, cnt1_hbm.at[pl.ds(s * TPT, TPT)])


@functools.lru_cache(maxsize=None)
def _prep_kernel():
    mesh = plsc.VectorSubcoreMesh(core_axis_name="c", subcore_axis_name="s")
    return pl.kernel(
        _prep_body,
        out_type=(jax.ShapeDtypeStruct((HACC,), jnp.float32),
                  jax.ShapeDtypeStruct((HACC,), jnp.float32),
                  jax.ShapeDtypeStruct((EPAD,), jnp.int32)),
        mesh=mesh,
        scratch_types=[
            pltpu.VMEM((CCH,), jnp.int32),
            pltpu.VMEM((CCH,), jnp.int32),
            pltpu.VMEM((CCH,), jnp.int32),
            pltpu.VMEM((CCH,), jnp.float32),
            pltpu.VMEM((TPT,), jnp.float32),
            pltpu.VMEM_SHARED((SACC,), jnp.float32),
        ],
        compiler_params=pltpu.CompilerParams(use_tc_tiling_on_sc=False),
    )


def _agg_body(pk_hbm, h2_hbm, a0_hbm, a1_hbm, ib, rows, sems, acc):
    c = lax.axis_index("c")
    s = lax.axis_index("s")
    zz = jnp.zeros((16,), jnp.float32)
    def zrows(i, carry):
        for j in range(HD // 16):
            rows[0, i, pl.ds(j * 16, 16)] = zz
        return carry
    lax.fori_loop(0, ECH, zrows, 0)
    for m in range(TPT // ECH):
        pltpu.sync_copy(rows.at[0], acc.at[pl.ds(s * TPT + m * ECH, ECH)])
    pltpu.sync_copy(rows.at[0], acc.at[pl.ds(s * TPT + TPT - ECH, ECH)])
    plsc.subcore_barrier()

    def load_idx(k, m):
        pltpu.async_copy(pk_hbm.at[s, k], ib.at[m], sems[m])

    def wait_idx(m):
        pltpu.make_async_copy(pk_hbm.at[s, 0], ib.at[m], sems[m]).wait()

    def xform_fire(b, m):
        for j in range(ECH // 16):
            sl = pl.ds(j * 16, 16)
            ib[m, 0, sl] = ib[m, 0, sl] * 2 + c
        pltpu.async_copy(h2_hbm.at[ib.at[m, 0]], rows.at[b], sems[4 + b])

    def wait_gather(b):
        pltpu.make_async_copy(h2_hbm.at[ib.at[0, 0]], rows.at[b],
                              sems[4 + b]).wait()

    for m in range(4):
        load_idx(m, m)
    wait_idx(0)
    xform_fire(0, 0)
    wait_idx(1)
    xform_fire(1, 1)

    def body(k4, carry):
        for b in range(4):
            k = k4 * 4 + b
            wait_gather(b % 2)
            pltpu.sync_copy(rows.at[b % 2], acc.at[ib.at[b, 1]], add=True)
            @pl.when(k + 4 < CHUNKS)
            def _():
                load_idx(k + 4, b)
            @pl.when(k + 2 < CHUNKS)
            def _():
                wait_idx((b + 2) % 4)
                xform_fire(b % 2, (b + 2) % 4)
        return carry
    lax.fori_loop(0, CHUNKS // 4, body, 0)
    plsc.subcore_barrier()

    @pl.when(c == 0)
    def _out0():
        pltpu.sync_copy(acc.at[pl.ds(s * TPT, TPT)],
                        a0_hbm.at[pl.ds(s * TPT, TPT)])

    @pl.when(c == 1)
    def _out1():
        pltpu.sync_copy(acc.at[pl.ds(s * TPT, TPT)],
                        a1_hbm.at[pl.ds(s * TPT, TPT)])


@functools.lru_cache(maxsize=None)
def _agg_kernel():
    mesh = plsc.VectorSubcoreMesh(core_axis_name="c", subcore_axis_name="s")
    return pl.kernel(
        _agg_body,
        out_type=(jax.ShapeDtypeStruct((HACC, HD), jnp.float32),
                  jax.ShapeDtypeStruct((HACC, HD), jnp.float32)),
        mesh=mesh,
        scratch_types=[
            pltpu.VMEM((4, 2, ECH), jnp.int32),
            pltpu.VMEM((NBUF, ECH, HD), jnp.float32),
            [pltpu.SemaphoreType.DMA] * 6,
            pltpu.VMEM_SHARED((SACC, HD), jnp.float32),
        ],
        compiler_params=pltpu.CompilerParams(use_tc_tiling_on_sc=False),
    )


def _proj_body(x_ref, w_ref, b_ref, o_ref):
    o_ref[...] = (jnp.dot(x_ref[...], w_ref[...],
                          preferred_element_type=jnp.float32) + b_ref[...])


_proj = pl.pallas_call(
    _proj_body, grid=(N // PBN,),
    in_specs=[pl.BlockSpec((PBN, D), lambda i: (i, 0)),
              pl.BlockSpec((D, D), lambda i: (0, 0)),
              pl.BlockSpec((1, D), lambda i: (0, 0))],
    out_specs=pl.BlockSpec((PBN, D), lambda i: (i, 0)),
    out_shape=jax.ShapeDtypeStruct((N, D), jnp.float32),
)


def _combine_body(h_ref, a0_ref, a1_ref, c0_ref, c1_ref, rw_ref, w0_ref,
                  w1_ref, cb_ref, g_ref, b_ref, o_ref):
    h = h_ref[...]
    inv = 1.0 / jnp.maximum(c0_ref[0] + c1_ref[0], 1.0)
    out = (jnp.dot(h, rw_ref[...], preferred_element_type=jnp.float32)
           + cb_ref[...])
    sc = jnp.concatenate(
        [jnp.broadcast_to(inv[:, r:r + 1], (BN, HD)) for r in range(R)],
        axis=1)
    out = out + jnp.dot(a0_ref[...] * sc, w0_ref[...],
                        preferred_element_type=jnp.float32)
    out = out + jnp.dot(a1_ref[...] * sc, w1_ref[...],
                        preferred_element_type=jnp.float32)
    z = jnp.maximum(out + h, 0.0)
    mu = jnp.mean(z, axis=-1, keepdims=True)
    zc = z - mu
    var = jnp.mean(zc * zc, axis=-1, keepdims=True)
    o_ref[...] = zc * lax.rsqrt(var + 1e-5) * g_ref[...] + b_ref[...]


_combine = pl.pallas_call(
    _combine_body, grid=(NPAD // BN,),
    in_specs=[pl.BlockSpec((BN, D), lambda i: (i, 0)),
              pl.BlockSpec((BN, R * HD), lambda i: (i, 0)),
              pl.BlockSpec((BN, R * HD), lambda i: (i, 0)),
              pl.BlockSpec((1, BN, R), lambda i: (i, 0, 0)),
              pl.BlockSpec((1, BN, R), lambda i: (i, 0, 0)),
              pl.BlockSpec((D, D), lambda i: (0, 0)),
              pl.BlockSpec((R * HD, D), lambda i: (0, 0)),
              pl.BlockSpec((R * HD, D), lambda i: (0, 0)),
              pl.BlockSpec((1, D), lambda i: (0, 0)),
              pl.BlockSpec((1, D), lambda i: (0, 0)),
              pl.BlockSpec((1, D), lambda i: (0, 0))],
    out_specs=pl.BlockSpec((BN, D), lambda i: (i, 0)),
    out_shape=jax.ShapeDtypeStruct((N, D), jnp.float32),
)


def _pool_body(h_ref, b_ref, w1_ref, b1_ref, w2_ref, b2_ref, o_ref, hg):
    i = pl.program_id(0)

    @pl.when(i == 0)
    def _init():
        hg[...] = jnp.full((NGRAPH, D), -jnp.inf, jnp.float32)

    bb = b_ref[...]
    h = h_ref[...]
    for g in range(NGRAPH):
        m = jnp.max(jnp.where(bb == g, h, -jnp.inf), axis=0,
                    keepdims=True)
        hg[pl.ds(g, 1)] = jnp.maximum(hg[pl.ds(g, 1)], m)

    @pl.when(i == N // PBN - 1)
    def _head():
        hc = jnp.maximum(
            jnp.dot(hg[...], w1_ref[...], preferred_element_type=jnp.float32)
            + b1_ref[...], 0.0)
        o_ref[...] = (jnp.dot(hc, w2_ref[...],
                              preferred_element_type=jnp.float32)
                      + b2_ref[...])


_pool = pl.pallas_call(
    _pool_body, grid=(N // PBN,),
    in_specs=[pl.BlockSpec((PBN, D), lambda i: (i, 0)),
              pl.BlockSpec((PBN, 1), lambda i: (i, 0)),
              pl.BlockSpec((D, D), lambda i: (0, 0)),
              pl.BlockSpec((1, D), lambda i: (0, 0)),
              pl.BlockSpec((D, 4), lambda i: (0, 0)),
              pl.BlockSpec((1, 4), lambda i: (0, 0))],
    out_specs=pl.BlockSpec((NGRAPH, 4), lambda i: (0, 0)),
    out_shape=jax.ShapeDtypeStruct((NGRAPH, 4), jnp.float32),
    scratch_shapes=[pltpu.VMEM((NGRAPH, D), jnp.float32)],
)


def kernel(x, edge_index, edge_type, batch, params):
    src = edge_index[0]
    dst = edge_index[1]
    src_pad = jnp.concatenate([src, jnp.zeros((EPAD - E,), jnp.int32)])
    cnt0, cnt1, dst3 = _prep_kernel()(dst, edge_type)
    c03 = cnt0.reshape(NPAD // BN, BN, R)
    c13 = cnt1.reshape(NPAD // BN, BN, R)
    pk = jnp.stack([src_pad.reshape(NSUB, CHUNKS, ECH),
                    dst3.reshape(NSUB, CHUNKS, ECH)], axis=2)
    h = _proj(x, params['in_W'], params['in_b'].reshape(1, D))
    for i in range(3):
        a0, a1 = _agg_kernel()(pk, h.reshape(2 * N, HD))
        relw = params['rel_W'][i]
        w0 = relw[:, :HD, :].reshape(R * HD, D)
        w1 = relw[:, HD:, :].reshape(R * HD, D)
        h = _combine(h, a0.reshape(NPAD, R * HD), a1.reshape(NPAD, R * HD),
                     c03, c13, params['root_W'][i], w0, w1,
                     params['conv_b'][i].reshape(1, D),
                     params['ln_g'][i].reshape(1, D),
                     params['ln_b'][i].reshape(1, D))
    return _pool(h, batch.reshape(N, 1), params['cls_W1'],
                 params['cls_b1'].reshape(1, D), params['cls_W2'],
                 params['cls_b2'].reshape(1, 4))


# speedup vs baseline: 1.6713x; 1.6713x over previous
import functools

import jax
import jax.numpy as jnp
from jax import lax
from jax.experimental import pallas as pl
from jax.experimental.pallas import tpu as pltpu
from jax.experimental.pallas import tpu_sc as plsc

N = 10000
E = 320000
D = 128
R = 3
NGRAPH = 16
NSUB = 16
NW = 32
SACC = 30208
HACC = 30720
TPT = SACC // NSUB
SPLIT = 15000
LACC = 15104
TPTL = LACC // NSUB
ECH = 32
CAP = 10240
PKB = 10368
EW = E // NW
CCH = 80
CNCH = EW // CCH
ZROW = N
SHIFT = 16384
NPAD = 10240
BN = 512

_SC_PARAMS = pltpu.CompilerParams(use_tc_tiling_on_sc=False,
                                  needs_layout_passes=False)


def _prep_body(src_hbm, dst_hbm, et_hbm, cnt0_hbm, cnt1_hbm, pk_hbm, cw_hbm,
               sb, db, eb, ob, onesb, sumbuf, pkbuf, cwb, spm):
    c = lax.axis_index("c")
    s = lax.axis_index("s")
    w = c * NSUB + s
    zz = jnp.zeros((16,), jnp.float32)

    def zero(i, carry):
        sumbuf[pl.ds(i * 16, 16)] = zz
        return carry
    lax.fori_loop(0, TPT // 16, zero, 0)
    pltpu.sync_copy(sumbuf, spm.at[pl.ds(s * TPT, TPT)])
    ones = jnp.ones((16,), jnp.float32)
    for j in range(CCH // 16):
        onesb[pl.ds(j * 16, 16)] = ones
    plsc.subcore_barrier()

    def chunk(k, offs):
        off0, off1 = offs
        base = w * EW + k * CCH
        pltpu.sync_copy(src_hbm.at[pl.ds(base, CCH)], sb)
        pltpu.sync_copy(dst_hbm.at[pl.ds(base, CCH)], db)
        pltpu.sync_copy(et_hbm.at[pl.ds(base, CCH)], eb)
        for j in range(CCH // 16):
            sl = pl.ds(j * 16, 16)
            d3 = db[sl] * 3 + eb[sl]
            ob[sl] = d3
            sel0 = d3 < SPLIT
            local = d3 - jnp.where(sel0, 0, SPLIT)
            val = sb[sl] * SHIFT + local
            key = jnp.where(sel0, 0, 1)
            _, v0 = plsc.sort_key_val(key, val)
            pkbuf[0, pl.ds(off0, 16)] = v0
            _, v1 = plsc.sort_key_val(1 - key, val)
            pkbuf[1, pl.ds(off1, 16)] = v1
            n0 = jnp.sum(jnp.where(sel0, 1, 0))
            off0 = off0 + n0
            off1 = off1 + (16 - n0)
        pltpu.sync_copy(onesb, spm.at[ob], add=True)
        return (off0, off1)
    off0, off1 = lax.fori_loop(0, CNCH, chunk, (0, 0))

    padv = jnp.full((16,), ZROW * SHIFT, jnp.int32)
    for t in range(4):
        pkbuf[0, pl.ds(off0 + t * 16, 16)] = padv
        pkbuf[1, pl.ds(off1 + t * 16, 16)] = padv
    pltpu.sync_copy(pkbuf.at[0, pl.ds(0, CAP)], pk_hbm.at[0, w])
    pltpu.sync_copy(pkbuf.at[1, pl.ds(0, CAP)], pk_hbm.at[1, w])
    lane = lax.iota(jnp.int32, 16)
    cwb[...] = jnp.where(lane == 0, off0, jnp.where(lane == 1, off1, 0))
    pltpu.sync_copy(cwb, cw_hbm.at[w])

    plsc.subcore_barrier()
    pltpu.sync_copy(spm.at[pl.ds(s * TPT, TPT)], sumbuf)

    @pl.when(c == 0)
    def _c0():
        pltpu.sync_copy(sumbuf, cnt0_hbm.at[pl.ds(s * TPT, TPT)])

    @pl.when(c == 1)
    def _c1():
        pltpu.sync_copy(sumbuf, cnt1_hbm.at[pl.ds(s * TPT, TPT)])


@functools.lru_cache(maxsize=None)
def _prep_kernel():
    mesh = plsc.VectorSubcoreMesh(core_axis_name="c", subcore_axis_name="s")
    return pl.kernel(
        _prep_body,
        out_type=(jax.ShapeDtypeStruct((HACC,), jnp.float32),
                  jax.ShapeDtypeStruct((HACC,), jnp.float32),
                  jax.ShapeDtypeStruct((2, NW, CAP), jnp.int32),
                  jax.ShapeDtypeStruct((NW, 16), jnp.int32)),
        mesh=mesh,
        scratch_types=[
            pltpu.VMEM((CCH,), jnp.int32),
            pltpu.VMEM((CCH,), jnp.int32),
            pltpu.VMEM((CCH,), jnp.int32),
            pltpu.VMEM((CCH,), jnp.int32),
            pltpu.VMEM((CCH,), jnp.float32),
            pltpu.VMEM((TPT,), jnp.float32),
            pltpu.VMEM((2, PKB), jnp.int32),
            pltpu.VMEM((16,), jnp.int32),
            pltpu.VMEM_SHARED((SACC,), jnp.float32),
        ],
        compiler_params=_SC_PARAMS,
    )


def _agg_body(pk_hbm, cw_hbm, h_hbm, a_hbm, ib, siv, riv, rows, cwb, isem,
              gsem, acc):
    c = lax.axis_index("c")
    s = lax.axis_index("s")
    zz = jnp.zeros((16,), jnp.float32)

    def zrows(i, carry):
        for j in range(D // 16):
            rows[0, i, pl.ds(j * 16, 16)] = zz
        return carry
    lax.fori_loop(0, ECH, zrows, 0)
    for m in range(TPTL // ECH):
        pltpu.sync_copy(rows.at[0], acc.at[pl.ds(s * TPTL + m * ECH, ECH)])
    pltpu.sync_copy(rows.at[0], acc.at[pl.ds(s * TPTL + TPTL - ECH, ECH)])
    plsc.subcore_barrier()

    for slot in range(2):
        w = 2 * s + slot
        pltpu.sync_copy(cw_hbm.at[w], cwb)
        cnt = jnp.sum(jnp.where(lax.iota(jnp.int32, 16) == c, cwb[...], 0))
        nch = (cnt + ECH - 1) // ECH

        def fire_idx(k):
            m = lax.rem(k, 4)
            pltpu.async_copy(pk_hbm.at[c, w, pl.ds(k * ECH, ECH)],
                             ib.at[m], isem)

        def wait_idx():
            pltpu.make_async_copy(pk_hbm.at[c, 0, pl.ds(0, ECH)], ib.at[0],
                                  isem).wait()

        def unpack_fire(k):
            m = lax.rem(k, 4)
            b = lax.rem(k, 2)
            for j in range(ECH // 16):
                sl = pl.ds(j * 16, 16)
                v = ib[m, sl]
                siv[b, sl] = lax.shift_right_logical(v, 14)
                riv[b, sl] = lax.bitwise_and(v, SHIFT - 1)
            pltpu.async_copy(h_hbm.at[siv.at[b]], rows.at[b], gsem)

        def wait_gather(b):
            pltpu.make_async_copy(h_hbm.at[siv.at[0]], rows.at[b],
                                  gsem).wait()

        for i in range(4):
            @pl.when(i < nch)
            def _():
                fire_idx(i)
        for i in range(2):
            @pl.when(i < nch)
            def _():
                wait_idx()
                unpack_fire(i)

        def body(k, carry):
            b = lax.rem(k, 2)
            wait_gather(b)
            pltpu.sync_copy(rows.at[b], acc.at[riv.at[b]], add=True)

            @pl.when(k + 4 < nch)
            def _():
                fire_idx(k + 4)

            @pl.when(k + 2 < nch)
            def _():
                wait_idx()
                unpack_fire(k + 2)
            return carry
        lax.fori_loop(0, nch, body, 0)

    plsc.subcore_barrier()

    @pl.when(s < NSUB - 1)
    def _full():
        pltpu.sync_copy(acc.at[pl.ds(s * TPTL, TPTL)],
                        a_hbm.at[pl.ds(c * SPLIT + s * TPTL, TPTL)])

    @pl.when(s == NSUB - 1)
    def _last():
        nlast = SPLIT - (NSUB - 1) * TPTL
        pltpu.sync_copy(acc.at[pl.ds((NSUB - 1) * TPTL, nlast)],
                        a_hbm.at[pl.ds(c * SPLIT + (NSUB - 1) * TPTL,
                                       nlast)])


@functools.lru_cache(maxsize=None)
def _agg_kernel():
    mesh = plsc.VectorSubcoreMesh(core_axis_name="c", subcore_axis_name="s")
    return pl.kernel(
        _agg_body,
        out_type=jax.ShapeDtypeStruct((HACC, D), jnp.float32),
        mesh=mesh,
        scratch_types=[
            pltpu.VMEM((4, ECH), jnp.int32),
            pltpu.VMEM((2, ECH), jnp.int32),
            pltpu.VMEM((2, ECH), jnp.int32),
            pltpu.VMEM((2, ECH, D), jnp.float32),
            pltpu.VMEM((16,), jnp.int32),
            pltpu.SemaphoreType.DMA,
            pltpu.SemaphoreType.DMA,
            pltpu.VMEM_SHARED((LACC, D), jnp.float32),
        ],
        compiler_params=_SC_PARAMS,
    )


def _proj_body(x_ref, w_ref, b_ref, o_ref):
    i = pl.program_id(0)
    rid = i * BN + lax.broadcasted_iota(jnp.int32, (BN, 1), 0)
    v = (jnp.dot(x_ref[...], w_ref[...],
                 preferred_element_type=jnp.float32) + b_ref[...])
    o_ref[...] = jnp.where(rid < N, v, 0.0)


_proj = pl.pallas_call(
    _proj_body, grid=(NPAD // BN,),
    in_specs=[pl.BlockSpec((BN, D), lambda i: (i, 0)),
              pl.BlockSpec((D, D), lambda i: (0, 0)),
              pl.BlockSpec((1, D), lambda i: (0, 0))],
    out_specs=pl.BlockSpec((BN, D), lambda i: (i, 0)),
    out_shape=jax.ShapeDtypeStruct((NPAD, D), jnp.float32),
)


def _combine_body(h_ref, a_ref, c0_ref, c1_ref, rw_ref, wr_ref, cb_ref,
                  g_ref, b_ref, o_ref):
    i = pl.program_id(0)
    h = h_ref[...]
    inv = 1.0 / jnp.maximum(c0_ref[0] + c1_ref[0], 1.0)
    out = (jnp.dot(h, rw_ref[...], preferred_element_type=jnp.float32)
           + cb_ref[...])
    sc = jnp.concatenate(
        [jnp.broadcast_to(inv[:, r:r + 1], (BN, D)) for r in range(R)],
        axis=1)
    out = out + jnp.dot(a_ref[...] * sc, wr_ref[...],
                        preferred_element_type=jnp.float32)
    z = jnp.maximum(out + h, 0.0)
    mu = jnp.mean(z, axis=-1, keepdims=True)
    zc = z - mu
    var = jnp.mean(zc * zc, axis=-1, keepdims=True)
    v = zc * lax.rsqrt(var + 1e-5) * g_ref[...] + b_ref[...]
    rid = i * BN + lax.broadcasted_iota(jnp.int32, (BN, 1), 0)
    o_ref[...] = jnp.where(rid < N, v, 0.0)


_combine = pl.pallas_call(
    _combine_body, grid=(NPAD // BN,),
    in_specs=[pl.BlockSpec((BN, D), lambda i: (i, 0)),
              pl.BlockSpec((BN, R * D), lambda i: (i, 0)),
              pl.BlockSpec((1, BN, R), lambda i: (i, 0, 0)),
              pl.BlockSpec((1, BN, R), lambda i: (i, 0, 0)),
              pl.BlockSpec((D, D), lambda i: (0, 0)),
              pl.BlockSpec((R * D, D), lambda i: (0, 0)),
              pl.BlockSpec((1, D), lambda i: (0, 0)),
              pl.BlockSpec((1, D), lambda i: (0, 0)),
              pl.BlockSpec((1, D), lambda i: (0, 0))],
    out_specs=pl.BlockSpec((BN, D), lambda i: (i, 0)),
    out_shape=jax.ShapeDtypeStruct((NPAD, D), jnp.float32),
)


def _pool_body(h_ref, b_ref, w1_ref, b1_ref, w2_ref, b2_ref, o_ref, hg):
    i = pl.program_id(0)

    @pl.when(i == 0)
    def _init():
        hg[...] = jnp.full((NGRAPH, D), -jnp.inf, jnp.float32)

    bb = b_ref[...]
    h = h_ref[...]
    for g in range(NGRAPH):
        m = jnp.max(jnp.where(bb == g, h, -jnp.inf), axis=0, keepdims=True)
        hg[pl.ds(g, 1)] = jnp.maximum(hg[pl.ds(g, 1)], m)

    @pl.when(i == NPAD // BN - 1)
    def _head():
        hc = jnp.maximum(
            jnp.dot(hg[...], w1_ref[...], preferred_element_type=jnp.float32)
            + b1_ref[...], 0.0)
        o_ref[...] = (jnp.dot(hc, w2_ref[...],
                              preferred_element_type=jnp.float32)
                      + b2_ref[...])


_pool = pl.pallas_call(
    _pool_body, grid=(NPAD // BN,),
    in_specs=[pl.BlockSpec((BN, D), lambda i: (i, 0)),
              pl.BlockSpec((BN, 1), lambda i: (i, 0)),
              pl.BlockSpec((D, D), lambda i: (0, 0)),
              pl.BlockSpec((1, D), lambda i: (0, 0)),
              pl.BlockSpec((D, 4), lambda i: (0, 0)),
              pl.BlockSpec((1, 4), lambda i: (0, 0))],
    out_specs=pl.BlockSpec((NGRAPH, 4), lambda i: (0, 0)),
    out_shape=jax.ShapeDtypeStruct((NGRAPH, 4), jnp.float32),
    scratch_shapes=[pltpu.VMEM((NGRAPH, D), jnp.float32)],
)


def kernel(x, edge_index, edge_type, batch, params):
    src = edge_index[0]
    dst = edge_index[1]
    cnt0, cnt1, pk, cw = _prep_kernel()(src, dst, edge_type)
    c03 = cnt0.reshape(NPAD // BN, BN, R)
    c13 = cnt1.reshape(NPAD // BN, BN, R)
    h = _proj(x, params['in_W'], params['in_b'].reshape(1, D))
    for i in range(3):
        a = _agg_kernel()(pk, cw, h)
        h = _combine(h, a.reshape(NPAD, R * D), c03, c13,
                     params['root_W'][i], params['rel_W'][i].reshape(R * D, D),
                     params['conv_b'][i].reshape(1, D),
                     params['ln_g'][i].reshape(1, D),
                     params['ln_b'][i].reshape(1, D))
    batch_pad = jnp.concatenate(
        [batch, jnp.full((NPAD - N,), NGRAPH, jnp.int32)]).reshape(NPAD, 1)
    return _pool(h, batch_pad, params['cls_W1'],
                 params['cls_b1'].reshape(1, D), params['cls_W2'],
                 params['cls_b2'].reshape(1, 4))


# speedup vs baseline: 1.9050x; 1.1398x over previous
import functools

import jax
import jax.numpy as jnp
from jax import lax
from jax.experimental import pallas as pl
from jax.experimental.pallas import tpu as pltpu
from jax.experimental.pallas import tpu_sc as plsc

N = 10000
E = 320000
D = 128
R = 3
NGRAPH = 16
NSUB = 16
NW = 32
SACC = 30208
HACC = 30720
TPT = SACC // NSUB
SPLIT = 15000
LACC = 15104
TPTL = LACC // NSUB
ECH = 32
CAP = 10240
PKB = 10368
EW = E // NW
CCH = 80
CNCH = EW // CCH
ZROW = N
SHIFT = 16384
NPAD = 10240
BN = 512

_SC_PARAMS = pltpu.CompilerParams(use_tc_tiling_on_sc=False,
                                  needs_layout_passes=False)


def _prep_body(src_hbm, dst_hbm, et_hbm, cnt0_hbm, cnt1_hbm, pk_hbm, cw_hbm,
               sb, db, eb, ob, onesb, sumbuf, pkbuf, cwb, psem, ssem, spm):
    c = lax.axis_index("c")
    s = lax.axis_index("s")
    w = c * NSUB + s
    zz = jnp.zeros((16,), jnp.float32)

    def zero(i, carry):
        sumbuf[pl.ds(i * 16, 16)] = zz
        return carry
    lax.fori_loop(0, TPT // 16, zero, 0)
    pltpu.sync_copy(sumbuf, spm.at[pl.ds(s * TPT, TPT)])
    ones = jnp.ones((16,), jnp.float32)
    for j in range(CCH // 16):
        onesb[pl.ds(j * 16, 16)] = ones
    plsc.subcore_barrier()

    def fire_loads(k, b):
        base = w * EW + k * CCH
        pltpu.async_copy(src_hbm.at[pl.ds(base, CCH)], sb.at[b], psem)
        pltpu.async_copy(dst_hbm.at[pl.ds(base, CCH)], db.at[b], psem)
        pltpu.async_copy(et_hbm.at[pl.ds(base, CCH)], eb.at[b], psem)

    def wait_loads():
        for ref in (sb, db, eb):
            pltpu.make_async_copy(src_hbm.at[pl.ds(0, CCH)], ref.at[0],
                                  psem).wait()

    def wait_scatter():
        pltpu.make_async_copy(onesb, spm.at[pl.ds(0, CCH)], ssem).wait()

    fire_loads(0, 0)

    def chunk(k, offs):
        off0, off1 = offs
        b = lax.rem(k, 2)
        wait_loads()

        @pl.when(k + 1 < CNCH)
        def _():
            fire_loads(k + 1, 1 - b)
        for j in range(CCH // 16):
            sl = pl.ds(j * 16, 16)
            d3 = db[b, sl] * 3 + eb[b, sl]
            ob[b, sl] = d3
            sel0 = d3 < SPLIT
            local = d3 - jnp.where(sel0, 0, SPLIT)
            val = sb[b, sl] * SHIFT + local
            key = jnp.where(sel0, 0, 1)
            _, v0 = plsc.sort_key_val(key, val)
            pkbuf[0, pl.ds(off0, 16)] = v0
            _, v1 = plsc.sort_key_val(1 - key, val)
            pkbuf[1, pl.ds(off1, 16)] = v1
            n0 = jnp.sum(jnp.where(sel0, 1, 0))
            off0 = off0 + n0
            off1 = off1 + (16 - n0)

        @pl.when(k >= 1)
        def _():
            wait_scatter()
        pltpu.async_copy(onesb, spm.at[ob.at[b]], ssem, add=True)
        return (off0, off1)
    off0, off1 = lax.fori_loop(0, CNCH, chunk, (0, 0))
    wait_scatter()

    padv = jnp.full((16,), ZROW * SHIFT, jnp.int32)
    for t in range(4):
        pkbuf[0, pl.ds(off0 + t * 16, 16)] = padv
        pkbuf[1, pl.ds(off1 + t * 16, 16)] = padv
    pltpu.sync_copy(pkbuf.at[0, pl.ds(0, CAP)], pk_hbm.at[0, w])
    pltpu.sync_copy(pkbuf.at[1, pl.ds(0, CAP)], pk_hbm.at[1, w])
    lane = lax.iota(jnp.int32, 16)
    cwb[...] = jnp.where(lane == 0, off0, jnp.where(lane == 1, off1, 0))
    pltpu.sync_copy(cwb, cw_hbm.at[w])

    plsc.subcore_barrier()
    pltpu.sync_copy(spm.at[pl.ds(s * TPT, TPT)], sumbuf)

    @pl.when(c == 0)
    def _c0():
        pltpu.sync_copy(sumbuf, cnt0_hbm.at[pl.ds(s * TPT, TPT)])

    @pl.when(c == 1)
    def _c1():
        pltpu.sync_copy(sumbuf, cnt1_hbm.at[pl.ds(s * TPT, TPT)])


@functools.lru_cache(maxsize=None)
def _prep_kernel():
    mesh = plsc.VectorSubcoreMesh(core_axis_name="c", subcore_axis_name="s")
    return pl.kernel(
        _prep_body,
        out_type=(jax.ShapeDtypeStruct((HACC,), jnp.float32),
                  jax.ShapeDtypeStruct((HACC,), jnp.float32),
                  jax.ShapeDtypeStruct((2, NW, CAP), jnp.int32),
                  jax.ShapeDtypeStruct((NW, 16), jnp.int32)),
        mesh=mesh,
        scratch_types=[
            pltpu.VMEM((2, CCH), jnp.int32),
            pltpu.VMEM((2, CCH), jnp.int32),
            pltpu.VMEM((2, CCH), jnp.int32),
            pltpu.VMEM((2, CCH), jnp.int32),
            pltpu.VMEM((CCH,), jnp.float32),
            pltpu.VMEM((TPT,), jnp.float32),
            pltpu.VMEM((2, PKB), jnp.int32),
            pltpu.VMEM((16,), jnp.int32),
            pltpu.SemaphoreType.DMA,
            pltpu.SemaphoreType.DMA,
            pltpu.VMEM_SHARED((SACC,), jnp.float32),
        ],
        compiler_params=_SC_PARAMS,
    )


def _agg_body(pk_hbm, cw_hbm, h_hbm, a_hbm, ib, siv, riv, rows, cwb, isem,
              gsem, acc):
    c = lax.axis_index("c")
    s = lax.axis_index("s")
    zz = jnp.zeros((16,), jnp.float32)

    def zrows(i, carry):
        for j in range(D // 16):
            rows[0, i, pl.ds(j * 16, 16)] = zz
        return carry
    lax.fori_loop(0, ECH, zrows, 0)
    for m in range(TPTL // ECH):
        pltpu.sync_copy(rows.at[0], acc.at[pl.ds(s * TPTL + m * ECH, ECH)])
    pltpu.sync_copy(rows.at[0], acc.at[pl.ds(s * TPTL + TPTL - ECH, ECH)])
    plsc.subcore_barrier()

    for slot in range(2):
        w = 2 * s + slot
        pltpu.sync_copy(cw_hbm.at[w], cwb)
        cnt = jnp.sum(jnp.where(lax.iota(jnp.int32, 16) == c, cwb[...], 0))
        nch = (cnt + ECH - 1) // ECH

        def fire_idx(k):
            m = lax.rem(k, 4)
            pltpu.async_copy(pk_hbm.at[c, w, pl.ds(k * ECH, ECH)],
                             ib.at[m], isem)

        def wait_idx():
            pltpu.make_async_copy(pk_hbm.at[c, 0, pl.ds(0, ECH)], ib.at[0],
                                  isem).wait()

        def unpack_fire(k):
            m = lax.rem(k, 4)
            b = lax.rem(k, 2)
            for j in range(ECH // 16):
                sl = pl.ds(j * 16, 16)
                v = ib[m, sl]
                siv[b, sl] = lax.shift_right_logical(v, 14)
                riv[b, sl] = lax.bitwise_and(v, SHIFT - 1)
            pltpu.async_copy(h_hbm.at[siv.at[b]], rows.at[b], gsem)

        def wait_gather(b):
            pltpu.make_async_copy(h_hbm.at[siv.at[0]], rows.at[b],
                                  gsem).wait()

        for i in range(4):
            @pl.when(i < nch)
            def _():
                fire_idx(i)
        for i in range(2):
            @pl.when(i < nch)
            def _():
                wait_idx()
                unpack_fire(i)

        def body(k, carry):
            b = lax.rem(k, 2)
            wait_gather(b)
            pltpu.sync_copy(rows.at[b], acc.at[riv.at[b]], add=True)

            @pl.when(k + 4 < nch)
            def _():
                fire_idx(k + 4)

            @pl.when(k + 2 < nch)
            def _():
                wait_idx()
                unpack_fire(k + 2)
            return carry
        lax.fori_loop(0, nch, body, 0)

    plsc.subcore_barrier()

    @pl.when(s < NSUB - 1)
    def _full():
        pltpu.sync_copy(acc.at[pl.ds(s * TPTL, TPTL)],
                        a_hbm.at[pl.ds(c * SPLIT + s * TPTL, TPTL)])

    @pl.when(s == NSUB - 1)
    def _last():
        nlast = SPLIT - (NSUB - 1) * TPTL
        pltpu.sync_copy(acc.at[pl.ds((NSUB - 1) * TPTL, nlast)],
                        a_hbm.at[pl.ds(c * SPLIT + (NSUB - 1) * TPTL,
                                       nlast)])


@functools.lru_cache(maxsize=None)
def _agg_kernel():
    mesh = plsc.VectorSubcoreMesh(core_axis_name="c", subcore_axis_name="s")
    return pl.kernel(
        _agg_body,
        out_type=jax.ShapeDtypeStruct((HACC, D), jnp.float32),
        mesh=mesh,
        scratch_types=[
            pltpu.VMEM((4, ECH), jnp.int32),
            pltpu.VMEM((2, ECH), jnp.int32),
            pltpu.VMEM((2, ECH), jnp.int32),
            pltpu.VMEM((2, ECH, D), jnp.float32),
            pltpu.VMEM((16,), jnp.int32),
            pltpu.SemaphoreType.DMA,
            pltpu.SemaphoreType.DMA,
            pltpu.VMEM_SHARED((LACC, D), jnp.float32),
        ],
        compiler_params=_SC_PARAMS,
    )


def _proj_body(x_ref, w_ref, b_ref, o_ref):
    i = pl.program_id(0)
    rid = i * BN + lax.broadcasted_iota(jnp.int32, (BN, 1), 0)
    v = (jnp.dot(x_ref[...], w_ref[...],
                 preferred_element_type=jnp.float32) + b_ref[...])
    o_ref[...] = jnp.where(rid < N, v, 0.0)


_proj = pl.pallas_call(
    _proj_body, grid=(NPAD // BN,),
    in_specs=[pl.BlockSpec((BN, D), lambda i: (i, 0)),
              pl.BlockSpec((D, D), lambda i: (0, 0)),
              pl.BlockSpec((1, D), lambda i: (0, 0))],
    out_specs=pl.BlockSpec((BN, D), lambda i: (i, 0)),
    out_shape=jax.ShapeDtypeStruct((NPAD, D), jnp.float32),
)


def _combine_body(h_ref, a_ref, c0_ref, c1_ref, rw_ref, wr_ref, cb_ref,
                  g_ref, b_ref, o_ref):
    i = pl.program_id(0)
    h = h_ref[...]
    inv = 1.0 / jnp.maximum(c0_ref[0] + c1_ref[0], 1.0)
    out = (jnp.dot(h, rw_ref[...], preferred_element_type=jnp.float32)
           + cb_ref[...])
    sc = jnp.concatenate(
        [jnp.broadcast_to(inv[:, r:r + 1], (BN, D)) for r in range(R)],
        axis=1)
    out = out + jnp.dot(a_ref[...] * sc, wr_ref[...],
                        preferred_element_type=jnp.float32)
    z = jnp.maximum(out + h, 0.0)
    mu = jnp.mean(z, axis=-1, keepdims=True)
    zc = z - mu
    var = jnp.mean(zc * zc, axis=-1, keepdims=True)
    v = zc * lax.rsqrt(var + 1e-5) * g_ref[...] + b_ref[...]
    rid = i * BN + lax.broadcasted_iota(jnp.int32, (BN, 1), 0)
    o_ref[...] = jnp.where(rid < N, v, 0.0)


_combine = pl.pallas_call(
    _combine_body, grid=(NPAD // BN,),
    in_specs=[pl.BlockSpec((BN, D), lambda i: (i, 0)),
              pl.BlockSpec((BN, R * D), lambda i: (i, 0)),
              pl.BlockSpec((1, BN, R), lambda i: (i, 0, 0)),
              pl.BlockSpec((1, BN, R), lambda i: (i, 0, 0)),
              pl.BlockSpec((D, D), lambda i: (0, 0)),
              pl.BlockSpec((R * D, D), lambda i: (0, 0)),
              pl.BlockSpec((1, D), lambda i: (0, 0)),
              pl.BlockSpec((1, D), lambda i: (0, 0)),
              pl.BlockSpec((1, D), lambda i: (0, 0))],
    out_specs=pl.BlockSpec((BN, D), lambda i: (i, 0)),
    out_shape=jax.ShapeDtypeStruct((NPAD, D), jnp.float32),
)


def _pool_body(h_ref, b_ref, w1_ref, b1_ref, w2_ref, b2_ref, o_ref, hg):
    i = pl.program_id(0)

    @pl.when(i == 0)
    def _init():
        hg[...] = jnp.full((NGRAPH, D), -jnp.inf, jnp.float32)

    bb = b_ref[...]
    h = h_ref[...]
    for g in range(NGRAPH):
        m = jnp.max(jnp.where(bb == g, h, -jnp.inf), axis=0, keepdims=True)
        hg[pl.ds(g, 1)] = jnp.maximum(hg[pl.ds(g, 1)], m)

    @pl.when(i == NPAD // BN - 1)
    def _head():
        hc = jnp.maximum(
            jnp.dot(hg[...], w1_ref[...], preferred_element_type=jnp.float32)
            + b1_ref[...], 0.0)
        o_ref[...] = (jnp.dot(hc, w2_ref[...],
                              preferred_element_type=jnp.float32)
                      + b2_ref[...])


_pool = pl.pallas_call(
    _pool_body, grid=(NPAD // BN,),
    in_specs=[pl.BlockSpec((BN, D), lambda i: (i, 0)),
              pl.BlockSpec((BN, 1), lambda i: (i, 0)),
              pl.BlockSpec((D, D), lambda i: (0, 0)),
              pl.BlockSpec((1, D), lambda i: (0, 0)),
              pl.BlockSpec((D, 4), lambda i: (0, 0)),
              pl.BlockSpec((1, 4), lambda i: (0, 0))],
    out_specs=pl.BlockSpec((NGRAPH, 4), lambda i: (0, 0)),
    out_shape=jax.ShapeDtypeStruct((NGRAPH, 4), jnp.float32),
    scratch_shapes=[pltpu.VMEM((NGRAPH, D), jnp.float32)],
)


def kernel(x, edge_index, edge_type, batch, params):
    src = edge_index[0]
    dst = edge_index[1]
    cnt0, cnt1, pk, cw = _prep_kernel()(src, dst, edge_type)
    c03 = cnt0.reshape(NPAD // BN, BN, R)
    c13 = cnt1.reshape(NPAD // BN, BN, R)
    h = _proj(x, params['in_W'], params['in_b'].reshape(1, D))
    for i in range(3):
        a = _agg_kernel()(pk, cw, h)
        h = _combine(h, a.reshape(NPAD, R * D), c03, c13,
                     params['root_W'][i], params['rel_W'][i].reshape(R * D, D),
                     params['conv_b'][i].reshape(1, D),
                     params['ln_g'][i].reshape(1, D),
                     params['ln_b'][i].reshape(1, D))
    batch_pad = jnp.concatenate(
        [batch, jnp.full((NPAD - N,), NGRAPH, jnp.int32)]).reshape(NPAD, 1)
    return _pool(h, batch_pad, params['cls_W1'],
                 params['cls_b1'].reshape(1, D), params['cls_W2'],
                 params['cls_b2'].reshape(1, 4))


# speedup vs baseline: 1.9137x; 1.0045x over previous
import functools

import jax
import jax.numpy as jnp
from jax import lax
from jax.experimental import pallas as pl
from jax.experimental.pallas import tpu as pltpu
from jax.experimental.pallas import tpu_sc as plsc

N = 10000
E = 320000
D = 128
R = 3
NGRAPH = 16
NSUB = 16
NW = 32
SACC = 30208
HACC = 30720
TPT = SACC // NSUB
SPLIT = 15000
LACC = 15104
TPTL = LACC // NSUB
ECH = 32
CAP = 10240
PKB = 10368
EW = E // NW
CCH = 80
CNCH = EW // CCH
ZROW = N
SHIFT = 16384
NPAD = 10240
BN = 512

_SC_PARAMS = pltpu.CompilerParams(use_tc_tiling_on_sc=False,
                                  needs_layout_passes=False)


def _prep_body(src_hbm, dst_hbm, et_hbm, cnt0_hbm, cnt1_hbm, pk_hbm, cw_hbm,
               sb, db, eb, ob, onesb, sumbuf, pkbuf, cwb, psem, ssem, spm):
    c = lax.axis_index("c")
    s = lax.axis_index("s")
    w = c * NSUB + s
    zz = jnp.zeros((16,), jnp.float32)

    def zero(i, carry):
        sumbuf[pl.ds(i * 16, 16)] = zz
        return carry
    lax.fori_loop(0, TPT // 16, zero, 0)
    pltpu.sync_copy(sumbuf, spm.at[pl.ds(s * TPT, TPT)])
    ones = jnp.ones((16,), jnp.float32)
    for j in range(CCH // 16):
        onesb[pl.ds(j * 16, 16)] = ones
    plsc.subcore_barrier()

    def fire_loads(k, b):
        base = w * EW + k * CCH
        pltpu.async_copy(src_hbm.at[pl.ds(base, CCH)], sb.at[b], psem)
        pltpu.async_copy(dst_hbm.at[pl.ds(base, CCH)], db.at[b], psem)
        pltpu.async_copy(et_hbm.at[pl.ds(base, CCH)], eb.at[b], psem)

    def wait_loads():
        for ref in (sb, db, eb):
            pltpu.make_async_copy(src_hbm.at[pl.ds(0, CCH)], ref.at[0],
                                  psem).wait()

    def wait_scatter():
        pltpu.make_async_copy(onesb, spm.at[pl.ds(0, CCH)], ssem).wait()

    fire_loads(0, 0)

    def chunk(k, offs):
        off0, off1 = offs
        b = lax.rem(k, 2)
        wait_loads()

        @pl.when(k + 1 < CNCH)
        def _():
            fire_loads(k + 1, 1 - b)
        for j in range(CCH // 16):
            sl = pl.ds(j * 16, 16)
            d3 = db[b, sl] * 3 + eb[b, sl]
            ob[b, sl] = d3
            sel0 = d3 < SPLIT
            local = d3 - jnp.where(sel0, 0, SPLIT)
            val = sb[b, sl] * SHIFT + local
            key = jnp.where(sel0, 0, 1)
            _, v0 = plsc.sort_key_val(key, val)
            pkbuf[0, pl.ds(off0, 16)] = v0
            _, v1 = plsc.sort_key_val(1 - key, val)
            pkbuf[1, pl.ds(off1, 16)] = v1
            n0 = jnp.sum(jnp.where(sel0, 1, 0))
            off0 = off0 + n0
            off1 = off1 + (16 - n0)

        @pl.when(k >= 1)
        def _():
            wait_scatter()
        pltpu.async_copy(onesb, spm.at[ob.at[b]], ssem, add=True)
        return (off0, off1)
    off0, off1 = lax.fori_loop(0, CNCH, chunk, (0, 0))
    wait_scatter()

    padv = jnp.full((16,), ZROW * SHIFT, jnp.int32)
    for t in range(4):
        pkbuf[0, pl.ds(off0 + t * 16, 16)] = padv
        pkbuf[1, pl.ds(off1 + t * 16, 16)] = padv
    pltpu.sync_copy(pkbuf.at[0, pl.ds(0, CAP)], pk_hbm.at[0, w])
    pltpu.sync_copy(pkbuf.at[1, pl.ds(0, CAP)], pk_hbm.at[1, w])
    lane = lax.iota(jnp.int32, 16)
    cwb[...] = jnp.where(lane == 0, off0, jnp.where(lane == 1, off1, 0))
    pltpu.sync_copy(cwb, cw_hbm.at[w])

    plsc.subcore_barrier()
    pltpu.sync_copy(spm.at[pl.ds(s * TPT, TPT)], sumbuf)

    @pl.when(c == 0)
    def _c0():
        pltpu.sync_copy(sumbuf, cnt0_hbm.at[pl.ds(s * TPT, TPT)])

    @pl.when(c == 1)
    def _c1():
        pltpu.sync_copy(sumbuf, cnt1_hbm.at[pl.ds(s * TPT, TPT)])


@functools.lru_cache(maxsize=None)
def _prep_kernel():
    mesh = plsc.VectorSubcoreMesh(core_axis_name="c", subcore_axis_name="s")
    return pl.kernel(
        _prep_body,
        out_type=(jax.ShapeDtypeStruct((HACC,), jnp.float32),
                  jax.ShapeDtypeStruct((HACC,), jnp.float32),
                  jax.ShapeDtypeStruct((2, NW, CAP), jnp.int32),
                  jax.ShapeDtypeStruct((NW, 16), jnp.int32)),
        mesh=mesh,
        scratch_types=[
            pltpu.VMEM((2, CCH), jnp.int32),
            pltpu.VMEM((2, CCH), jnp.int32),
            pltpu.VMEM((2, CCH), jnp.int32),
            pltpu.VMEM((2, CCH), jnp.int32),
            pltpu.VMEM((CCH,), jnp.float32),
            pltpu.VMEM((TPT,), jnp.float32),
            pltpu.VMEM((2, PKB), jnp.int32),
            pltpu.VMEM((16,), jnp.int32),
            pltpu.SemaphoreType.DMA,
            pltpu.SemaphoreType.DMA,
            pltpu.VMEM_SHARED((SACC,), jnp.float32),
        ],
        compiler_params=_SC_PARAMS,
    )


def _agg_body(pk_hbm, cw_hbm, h_hbm, a_hbm, ib, siv, riv, rows, cwb, isem,
              gsem, acc):
    c = lax.axis_index("c")
    s = lax.axis_index("s")
    zz = jnp.zeros((16,), jnp.float32)

    def zrows(i, carry):
        for j in range(D // 16):
            rows[0, i, pl.ds(j * 16, 16)] = zz
        return carry
    lax.fori_loop(0, ECH, zrows, 0)
    for m in range(TPTL // ECH):
        pltpu.sync_copy(rows.at[0], acc.at[pl.ds(s * TPTL + m * ECH, ECH)])
    pltpu.sync_copy(rows.at[0], acc.at[pl.ds(s * TPTL + TPTL - ECH, ECH)])
    plsc.subcore_barrier()

    for slot in range(2):
        w = 2 * s + slot
        pltpu.sync_copy(cw_hbm.at[w], cwb)
        cnt = jnp.sum(jnp.where(lax.iota(jnp.int32, 16) == c, cwb[...], 0))
        nch = (cnt + ECH - 1) // ECH

        def fire_idx(k):
            m = lax.rem(k, 4)
            pltpu.async_copy(pk_hbm.at[c, w, pl.ds(k * ECH, ECH)],
                             ib.at[m], isem)

        def wait_idx():
            pltpu.make_async_copy(pk_hbm.at[c, 0, pl.ds(0, ECH)], ib.at[0],
                                  isem).wait()

        def unpack_fire(k):
            m = lax.rem(k, 4)
            b = lax.rem(k, 2)
            for j in range(ECH // 16):
                sl = pl.ds(j * 16, 16)
                v = ib[m, sl]
                siv[b, sl] = lax.shift_right_logical(v, 14)
                riv[b, sl] = lax.bitwise_and(v, SHIFT - 1)
            pltpu.async_copy(h_hbm.at[siv.at[b]], rows.at[b], gsem)

        def wait_gather(b):
            pltpu.make_async_copy(h_hbm.at[siv.at[0]], rows.at[b],
                                  gsem).wait()

        for i in range(4):
            @pl.when(i < nch)
            def _():
                fire_idx(i)
        for i in range(2):
            @pl.when(i < nch)
            def _():
                wait_idx()
                unpack_fire(i)

        def body(k, carry):
            b = lax.rem(k, 2)
            wait_gather(b)
            pltpu.sync_copy(rows.at[b], acc.at[riv.at[b]], add=True)

            @pl.when(k + 4 < nch)
            def _():
                fire_idx(k + 4)

            @pl.when(k + 2 < nch)
            def _():
                wait_idx()
                unpack_fire(k + 2)
            return carry
        lax.fori_loop(0, nch, body, 0)

    plsc.subcore_barrier()

    @pl.when(s < NSUB - 1)
    def _full():
        pltpu.sync_copy(acc.at[pl.ds(s * TPTL, TPTL)],
                        a_hbm.at[pl.ds(c * SPLIT + s * TPTL, TPTL)])

    @pl.when(s == NSUB - 1)
    def _last():
        nlast = SPLIT - (NSUB - 1) * TPTL
        pltpu.sync_copy(acc.at[pl.ds((NSUB - 1) * TPTL, nlast)],
                        a_hbm.at[pl.ds(c * SPLIT + (NSUB - 1) * TPTL,
                                       nlast)])


@functools.lru_cache(maxsize=None)
def _agg_kernel():
    mesh = plsc.VectorSubcoreMesh(core_axis_name="c", subcore_axis_name="s")
    return pl.kernel(
        _agg_body,
        out_type=jax.ShapeDtypeStruct((HACC, D), jnp.float32),
        mesh=mesh,
        scratch_types=[
            pltpu.VMEM((4, ECH), jnp.int32),
            pltpu.VMEM((2, ECH), jnp.int32),
            pltpu.VMEM((2, ECH), jnp.int32),
            pltpu.VMEM((2, ECH, D), jnp.float32),
            pltpu.VMEM((16,), jnp.int32),
            pltpu.SemaphoreType.DMA,
            pltpu.SemaphoreType.DMA,
            pltpu.VMEM_SHARED((LACC, D), jnp.float32),
        ],
        compiler_params=_SC_PARAMS,
    )


def _proj_body(x_ref, w_ref, b_ref, o_ref):
    i = pl.program_id(0)
    rid = i * BN + lax.broadcasted_iota(jnp.int32, (BN, 1), 0)
    v = (jnp.dot(x_ref[...], w_ref[...],
                 preferred_element_type=jnp.float32) + b_ref[...])
    o_ref[...] = jnp.where(rid < N, v, 0.0)


_proj = pl.pallas_call(
    _proj_body, grid=(NPAD // BN,),
    in_specs=[pl.BlockSpec((BN, D), lambda i: (i, 0)),
              pl.BlockSpec((D, D), lambda i: (0, 0)),
              pl.BlockSpec((1, D), lambda i: (0, 0))],
    out_specs=pl.BlockSpec((BN, D), lambda i: (i, 0)),
    out_shape=jax.ShapeDtypeStruct((NPAD, D), jnp.float32),
)


def _combine_body(h_ref, a_ref, c0_ref, c1_ref, rw_ref, wr_ref, cb_ref,
                  g_ref, b_ref, o_ref):
    i = pl.program_id(0)
    h = h_ref[...]
    inv = 1.0 / jnp.maximum(c0_ref[0] + c1_ref[0], 1.0)
    out = (jnp.dot(h, rw_ref[...], preferred_element_type=jnp.float32)
           + cb_ref[...])
    sc = jnp.concatenate(
        [jnp.broadcast_to(inv[:, r:r + 1], (BN, D)) for r in range(R)],
        axis=1)
    out = out + jnp.dot(a_ref[...] * sc, wr_ref[...],
                        preferred_element_type=jnp.float32)
    z = jnp.maximum(out + h, 0.0)
    mu = jnp.mean(z, axis=-1, keepdims=True)
    zc = z - mu
    var = jnp.mean(zc * zc, axis=-1, keepdims=True)
    v = zc * lax.rsqrt(var + 1e-5) * g_ref[...] + b_ref[...]
    rid = i * BN + lax.broadcasted_iota(jnp.int32, (BN, 1), 0)
    o_ref[...] = jnp.where(rid < N, v, 0.0)


_combine = pl.pallas_call(
    _combine_body, grid=(NPAD // BN,),
    in_specs=[pl.BlockSpec((BN, D), lambda i: (i, 0)),
              pl.BlockSpec((BN, R * D), lambda i: (i, 0)),
              pl.BlockSpec((1, BN, R), lambda i: (i, 0, 0)),
              pl.BlockSpec((1, BN, R), lambda i: (i, 0, 0)),
              pl.BlockSpec((D, D), lambda i: (0, 0)),
              pl.BlockSpec((R * D, D), lambda i: (0, 0)),
              pl.BlockSpec((1, D), lambda i: (0, 0)),
              pl.BlockSpec((1, D), lambda i: (0, 0)),
              pl.BlockSpec((1, D), lambda i: (0, 0))],
    out_specs=pl.BlockSpec((BN, D), lambda i: (i, 0)),
    out_shape=jax.ShapeDtypeStruct((NPAD, D), jnp.float32),
)


def _combine3_body(h_ref, a_ref, c0_ref, c1_ref, rw_ref, wr_ref, cb_ref,
                   g_ref, b_ref, bt_ref, w1_ref, b1_ref, w2_ref, b2_ref,
                   o_ref, hg):
    i = pl.program_id(0)

    @pl.when(i == 0)
    def _init():
        hg[...] = jnp.full((NGRAPH, D), -jnp.inf, jnp.float32)

    h = h_ref[...]
    inv = 1.0 / jnp.maximum(c0_ref[0] + c1_ref[0], 1.0)
    out = (jnp.dot(h, rw_ref[...], preferred_element_type=jnp.float32)
           + cb_ref[...])
    sc = jnp.concatenate(
        [jnp.broadcast_to(inv[:, r:r + 1], (BN, D)) for r in range(R)],
        axis=1)
    out = out + jnp.dot(a_ref[...] * sc, wr_ref[...],
                        preferred_element_type=jnp.float32)
    z = jnp.maximum(out + h, 0.0)
    mu = jnp.mean(z, axis=-1, keepdims=True)
    zc = z - mu
    var = jnp.mean(zc * zc, axis=-1, keepdims=True)
    v = zc * lax.rsqrt(var + 1e-5) * g_ref[...] + b_ref[...]

    rid = i * BN + lax.broadcasted_iota(jnp.int32, (BN, 1), 0)
    bb = bt_ref[...]
    for g in range(NGRAPH):
        m = jnp.max(jnp.where((bb == g) & (rid < N), v, -jnp.inf), axis=0,
                    keepdims=True)
        hg[pl.ds(g, 1)] = jnp.maximum(hg[pl.ds(g, 1)], m)

    @pl.when(i == NPAD // BN - 1)
    def _head():
        hc = jnp.maximum(
            jnp.dot(hg[...], w1_ref[...], preferred_element_type=jnp.float32)
            + b1_ref[...], 0.0)
        o_ref[...] = (jnp.dot(hc, w2_ref[...],
                              preferred_element_type=jnp.float32)
                      + b2_ref[...])


_combine3 = pl.pallas_call(
    _combine3_body, grid=(NPAD // BN,),
    in_specs=[pl.BlockSpec((BN, D), lambda i: (i, 0)),
              pl.BlockSpec((BN, R * D), lambda i: (i, 0)),
              pl.BlockSpec((1, BN, R), lambda i: (i, 0, 0)),
              pl.BlockSpec((1, BN, R), lambda i: (i, 0, 0)),
              pl.BlockSpec((D, D), lambda i: (0, 0)),
              pl.BlockSpec((R * D, D), lambda i: (0, 0)),
              pl.BlockSpec((1, D), lambda i: (0, 0)),
              pl.BlockSpec((1, D), lambda i: (0, 0)),
              pl.BlockSpec((1, D), lambda i: (0, 0)),
              pl.BlockSpec((BN, 1), lambda i: (i, 0)),
              pl.BlockSpec((D, D), lambda i: (0, 0)),
              pl.BlockSpec((1, D), lambda i: (0, 0)),
              pl.BlockSpec((D, 4), lambda i: (0, 0)),
              pl.BlockSpec((1, 4), lambda i: (0, 0))],
    out_specs=pl.BlockSpec((NGRAPH, 4), lambda i: (0, 0)),
    out_shape=jax.ShapeDtypeStruct((NGRAPH, 4), jnp.float32),
    scratch_shapes=[pltpu.VMEM((NGRAPH, D), jnp.float32)],
)


def kernel(x, edge_index, edge_type, batch, params):
    src = edge_index[0]
    dst = edge_index[1]
    cnt0, cnt1, pk, cw = _prep_kernel()(src, dst, edge_type)
    c03 = cnt0.reshape(NPAD // BN, BN, R)
    c13 = cnt1.reshape(NPAD // BN, BN, R)
    h = _proj(x, params['in_W'], params['in_b'].reshape(1, D))
    for i in range(2):
        a = _agg_kernel()(pk, cw, h)
        h = _combine(h, a.reshape(NPAD, R * D), c03, c13,
                     params['root_W'][i], params['rel_W'][i].reshape(R * D, D),
                     params['conv_b'][i].reshape(1, D),
                     params['ln_g'][i].reshape(1, D),
                     params['ln_b'][i].reshape(1, D))
    a = _agg_kernel()(pk, cw, h)
    return _combine3(h, a.reshape(NPAD, R * D), c03, c13,
                     params['root_W'][2], params['rel_W'][2].reshape(R * D, D),
                     params['conv_b'][2].reshape(1, D),
                     params['ln_g'][2].reshape(1, D),
                     params['ln_b'][2].reshape(1, D),
                     batch.reshape(N, 1), params['cls_W1'],
                     params['cls_b1'].reshape(1, D), params['cls_W2'],
                     params['cls_b2'].reshape(1, 4))
